# baseline (speedup 1.0000x reference)
import functools

import jax
import jax.numpy as jnp
from jax import lax
from jax.experimental import pallas as pl
from jax.experimental.pallas import tpu as pltpu
from jax.experimental.pallas import tpu_sc as plsc

NC = 2
NS = 16
NW = NC * NS
L = 16

CB = 16
NQ = 4
NIR = 80
NOP = 24
JB = 8


def _tree_sum(vs):
    while len(vs) > 1:
        vs = [vs[i] + vs[i + 1] for i in range(0, len(vs) - 1, 2)] + (
            [vs[-1]] if len(vs) % 2 else []
        )
    return vs[0]


def _sc_scores_kernel(B, NEG, D):
    assert D == NQ * L
    assert B % (NW * CB) == 0
    b_per_w = B // NW
    n_chunks = b_per_w // CB
    assert n_chunks % 2 == 0
    neg_per_chunk = CB * NEG
    n_neg_dma = neg_per_chunk // NIR
    DP = 2 * D

    mesh = plsc.VectorSubcoreMesh(
        core_axis_name="c", subcore_axis_name="s", num_cores=NC, num_subcores=NS
    )

    @functools.partial(
        pl.kernel,
        out_type=(
            jax.ShapeDtypeStruct((B,), jnp.float32),
            jax.ShapeDtypeStruct((B,), jnp.float32),
        ),
        mesh=mesh,
        compiler_params=pltpu.CompilerParams(needs_layout_passes=False),
        scratch_types=dict(
            t_idxp=pltpu.VMEM((b_per_w,), jnp.int32),
            c_idxp=pltpu.VMEM((b_per_w,), jnp.int32),
            n_idxp=pltpu.VMEM((b_per_w * NEG,), jnp.int32),
            t_off=pltpu.VMEM((b_per_w,), jnp.int32),
            c_off=pltpu.VMEM((b_per_w,), jnp.int32),
            n_off=pltpu.VMEM((b_per_w * NOP,), jnp.int32),
            t_bufs=pltpu.VMEM((2, CB, DP), jnp.float32),
            c_bufs=pltpu.VMEM((2, CB, DP), jnp.float32),
            n_bufs=pltpu.VMEM((2, neg_per_chunk, DP), jnp.float32),
            pos_v=pltpu.VMEM((b_per_w,), jnp.float32),
            neg_v=pltpu.VMEM((b_per_w,), jnp.float32),
            semA=pltpu.SemaphoreType.DMA,
            semB=pltpu.SemaphoreType.DMA,
        ),
    )
    def sc_kernel(
        tgtp_hbm, ctxp_hbm, negp_hbm, toff_hbm, coff_hbm, noff_hbm,
        wt_hbm, wc_hbm,
        pos_hbm, negdot_hbm,
        t_idxp, c_idxp, n_idxp, t_off, c_off, n_off,
        t_bufs, c_bufs, n_bufs, pos_v, neg_v, semA, semB,
    ):
        wid = lax.axis_index("s") * NC + lax.axis_index("c")
        base = wid * b_per_w
        iota = lax.iota(jnp.int32, L)
        sems = (semA, semB)

        pltpu.sync_copy(tgtp_hbm.at[pl.ds(base, b_per_w)], t_idxp)
        pltpu.sync_copy(ctxp_hbm.at[pl.ds(base, b_per_w)], c_idxp)
        pltpu.sync_copy(negp_hbm.at[pl.ds(base * NEG, b_per_w * NEG)], n_idxp)
        pltpu.sync_copy(toff_hbm.at[pl.ds(base, b_per_w)], t_off)
        pltpu.sync_copy(coff_hbm.at[pl.ds(base, b_per_w)], c_off)
        pltpu.sync_copy(noff_hbm.at[pl.ds(base * NOP, b_per_w * NOP)], n_off)

        def fire(c, slot):
            sem = sems[slot]
            o = c * CB
            pltpu.async_copy(wt_hbm.at[t_idxp.at[pl.ds(o, CB)]], t_bufs.at[slot], sem)
            pltpu.async_copy(wc_hbm.at[c_idxp.at[pl.ds(o, CB)]], c_bufs.at[slot], sem)
            for r in range(n_neg_dma):
                pltpu.async_copy(
                    wc_hbm.at[n_idxp.at[pl.ds(o * NEG + r * NIR, NIR)]],
                    n_bufs.at[slot].at[pl.ds(r * NIR, NIR)],
                    sem,
                )

        def drain(slot):
            sem = sems[slot]
            pltpu.make_async_copy(wt_hbm.at[pl.ds(0, CB)], t_bufs.at[slot], sem).wait()
            pltpu.make_async_copy(wc_hbm.at[pl.ds(0, CB)], c_bufs.at[slot], sem).wait()
            pltpu.make_async_copy(
                wc_hbm.at[pl.ds(0, neg_per_chunk)], n_bufs.at[slot], sem
            ).wait()

        def compute(c, slot):
            tb = t_bufs.at[slot]
            cbuf = c_bufs.at[slot]
            nb = n_bufs.at[slot]
            o = c * CB

            def block(ib, carry):
                pos_acc, neg_acc = carry
                ob = o + ib * JB
                tov = t_off[pl.ds(ob, L)]
                cov = c_off[pl.ds(ob, L)]
                for jj in range(JB):
                    j = ib * JB + jj
                    th = tov[jj]
                    ch = cov[jj]
                    nov0 = n_off[pl.ds((ob + jj) * NOP, L)]
                    nov1 = n_off[pl.ds((ob + jj) * NOP + JB, L)]
                    t_q = [tb[j, pl.ds(th + q * L, L)] for q in range(NQ)]
                    c_q = [cbuf[j, pl.ds(ch + q * L, L)] for q in range(NQ)]
                    pos_s = jnp.sum(_tree_sum([t_q[q] * c_q[q] for q in range(NQ)]))
                    acc0 = [None] * NQ
                    acc1 = [None] * NQ
                    for k in range(NEG):
                        nh = nov0[k] if k < L else nov1[k - JB]
                        rj = j * NEG + k
                        acc = acc0 if k % 2 == 0 else acc1
                        for q in range(NQ):
                            v = nb[rj, pl.ds(nh + q * L, L)]
                            acc[q] = v if acc[q] is None else acc[q] + v
                    n_q = [acc0[q] + acc1[q] for q in range(NQ)]
                    neg_s = jnp.sum(_tree_sum([t_q[q] * n_q[q] for q in range(NQ)]))
                    pos_acc = jnp.where(iota == j, pos_s, pos_acc)
                    neg_acc = jnp.where(iota == j, neg_s, neg_acc)
                return pos_acc, neg_acc

            zero = jnp.zeros((L,), jnp.float32)
            pos_acc, neg_acc = lax.fori_loop(0, CB // JB, block, (zero, zero))
            pos_v[pl.ds(o, L)] = pos_acc
            neg_v[pl.ds(o, L)] = neg_acc

        fire(0, 0)
        fire(1, 1)

        def pair_body(i, carry):
            c0 = 2 * i
            drain(0)
            compute(c0, 0)

            @pl.when(c0 + 2 < n_chunks)
            def _():
                fire(c0 + 2, 0)

            drain(1)
            compute(c0 + 1, 1)

            @pl.when(c0 + 3 < n_chunks)
            def _():
                fire(c0 + 3, 1)

            return carry

        lax.fori_loop(0, n_chunks // 2, pair_body, 0)
        pltpu.sync_copy(pos_v, pos_hbm.at[pl.ds(base, b_per_w)])
        pltpu.sync_copy(neg_v, negdot_hbm.at[pl.ds(base, b_per_w)])

    return sc_kernel


def _tc_loss_kernel(pos_ref, neg_ref, out_ref):
    p = pos_ref[...]
    n = neg_ref[...]

    def ls(x):
        return jnp.minimum(x, 0.0) - jnp.log1p(jnp.exp(-jnp.abs(x)))

    out_ref[0, 0] = -(jnp.sum(ls(p)) + jnp.sum(ls(-n)))


def kernel(target_word, context_word, negative_example, W_target, W_context):
    B = target_word.shape[0]
    NEG = negative_example.shape[1]
    V, D = W_target.shape

    tgt = target_word.astype(jnp.int32)
    ctx = context_word.astype(jnp.int32)
    neg = negative_example.astype(jnp.int32)
    wt_p = W_target.reshape(V // 2, 2 * D)
    wc_p = W_context.reshape(V // 2, 2 * D)
    n_off = jnp.pad((neg & 1) * D, ((0, 0), (0, NOP - NEG))).reshape(B * NOP)

    sc = _sc_scores_kernel(B, NEG, D)
    pos_dot, neg_dot = sc(
        tgt >> 1, ctx >> 1, (neg >> 1).reshape(B * NEG),
        (tgt & 1) * D, (ctx & 1) * D, n_off,
        wt_p, wc_p,
    )

    r = B // 128
    loss = pl.pallas_call(
        _tc_loss_kernel,
        out_shape=jax.ShapeDtypeStruct((1, 1), jnp.float32),
        out_specs=pl.BlockSpec(memory_space=pltpu.SMEM),
    )(pos_dot.reshape(r, 128), neg_dot.reshape(r, 128))
    return loss[0, 0]
